# baseline (speedup 1.0000x reference)
import numpy as np
import jax
import jax.numpy as jnp
from jax import lax
from jax.experimental import pallas as pl
from jax.experimental.pallas import tpu as pltpu
from jax.experimental.pallas import tpu_sc as plsc

_PCX = np.float32(-51.2)
_PCY = np.float32(-51.2)
_VOXX = np.float32(0.1)
_VOXY = np.float32(0.1)
_B = 2
_C = 128
_H = 512
_W = 512
_K = 4096
_CPT = 8
_RB = 15
_LANES = 16
_STEPS = _K // _LANES
_BN_DIV = np.float32(np.sqrt(np.float32(1.0) + np.float32(1e-5)))


def _sc_body(cidx, ws, rr, rows_hbm, out_hbm, civ, wv, rrv, chunk, acc,
             sem, sem2):
    b = lax.axis_index("c")
    s = lax.axis_index("s")
    c0 = s * _CPT

    drr = pltpu.async_copy(rr.at[b], rrv, sem2)
    dci = pltpu.async_copy(cidx.at[b], civ, sem)
    dws = pltpu.async_copy(ws.at[b], wv, sem)
    drr.wait()
    rrvec = rrv[pl.ds(0, _LANES)]
    row_lo = rrvec[0]
    row_hi = rrvec[1]
    nch = (row_hi - row_lo + _RB) // _RB

    chunk_ch = [chunk.at[pl.ds(ch * _RB * _W, _RB * _W)] for ch in range(_CPT)]

    def issue_chunk(row_base):
        descs = []
        for ch in range(_CPT):
            for slot in range(_RB):
                descs.append(pltpu.async_copy(
                    rows_hbm.at[b, c0 + ch, row_base + slot, :],
                    chunk.at[pl.ds((ch * _RB + slot) * _W, _W)], sem2))
        return descs

    def fast_path():
        row_base = jnp.minimum(row_lo, _H - _RB)
        rb512 = row_base * _W
        descs = issue_chunk(row_base)
        dci.wait()
        dws.wait()
        for d in descs:
            d.wait()

        def substep(i):
            sl = pl.ds(i * _LANES, _LANES)
            accs = [None] * _CPT
            for c in range(4):
                base = civ[c, sl] - rb512
                w = wv[c, sl]
                for ch in range(_CPT):
                    v = plsc.load_gather(chunk_ch[ch], [base])
                    accs[ch] = v * w if accs[ch] is None else accs[ch] + v * w
            for ch in range(_CPT):
                acc[ch, sl] = accs[ch]

        @plsc.parallel_loop(0, _STEPS, unroll=2)
        def _steps(i):
            substep(i)

    def slow_path():
        dci.wait()
        dws.wait()
        zero = jnp.zeros((_LANES,), jnp.float32)

        def zstep(i, carry):
            for ch in range(_CPT):
                acc[ch, pl.ds(i * _LANES, _LANES)] = zero
            return carry

        lax.fori_loop(0, _STEPS, zstep, 0)

        def chunk_body(j, carry):
            chunk_lo = row_lo + j * _RB
            row_base = jnp.minimum(chunk_lo, _H - _RB)
            glo = chunk_lo * _W
            ghi = (chunk_lo + _RB) * _W
            rb512 = row_base * _W
            descs = issue_chunk(row_base)
            for d in descs:
                d.wait()

            def step(i, carry2):
                sl = pl.ds(i * _LANES, _LANES)
                accs = [acc[ch, sl] for ch in range(_CPT)]
                for c in range(4):
                    g = civ[c, sl]
                    w = wv[c, sl]
                    valid = (g >= glo) & (g < ghi)
                    base = jnp.clip(g - rb512, 0, _RB * _W - 1)
                    wm = jnp.where(valid, w, 0.0)
                    for ch in range(_CPT):
                        v = plsc.load_gather(chunk_ch[ch], [base])
                        accs[ch] = accs[ch] + wm * v
                for ch in range(_CPT):
                    acc[ch, sl] = accs[ch]
                return carry2

            lax.fori_loop(0, _STEPS, step, 0)
            return carry

        lax.fori_loop(0, nch, chunk_body, 0)

    lax.cond(nch == 1, fast_path, slow_path)
    pltpu.sync_copy(acc, out_hbm.at[b, pl.ds(c0, _CPT), :])


def _sc_gather(cidx, ws, rr, rows):
    fn = pl.kernel(
        _sc_body,
        out_type=jax.ShapeDtypeStruct((_B, _C, _K), jnp.float32),
        mesh=plsc.VectorSubcoreMesh(
            core_axis_name="c", subcore_axis_name="s",
            num_cores=2, num_subcores=16),
        scratch_types=[
            pltpu.VMEM((4, _K), jnp.int32),
            pltpu.VMEM((4, _K), jnp.float32),
            pltpu.VMEM((_LANES,), jnp.int32),
            pltpu.VMEM((_CPT * _RB * _W,), jnp.float32),
            pltpu.VMEM((_CPT, _K), jnp.float32),
            pltpu.SemaphoreType.DMA,
            pltpu.SemaphoreType.DMA,
        ],
        compiler_params=pltpu.CompilerParams(
            use_tc_tiling_on_sc=True, needs_layout_passes=False),
    )
    return fn(cidx, ws, rr, rows)


def _mm_body(ft_ref, w_ref, g_ref, bt_ref, o_ref):
    wmat = w_ref[...]
    g = g_ref[...]
    bt = bt_ref[...]
    for b in range(_B):
        ft = ft_ref[b]
        y = lax.dot_general(ft, wmat, (((0,), (1,)), ((), ())),
                            preferred_element_type=jnp.float32)
        y = y / _BN_DIV * g[None, :] + bt[None, :]
        o_ref[pl.ds(b * _K, _K), :] = jnp.maximum(y, 0.0)


def _mm_call(ft, W_fuse, gamma, beta):
    return pl.pallas_call(
        _mm_body,
        out_shape=jax.ShapeDtypeStruct((_B * _K, _C), jnp.float32),
    )(ft, W_fuse, gamma, beta)


def kernel(keypoints, spatial_features, W_fuse, gamma, beta, bev_stride):
    xi = (keypoints[:, :, 0] - _PCX) / _VOXX / bev_stride
    yi = (keypoints[:, :, 1] - _PCY) / _VOXY / bev_stride
    zi = keypoints[:, :, 2]

    x0 = jnp.floor(xi).astype(jnp.int32)
    y0 = jnp.floor(yi).astype(jnp.int32)
    x1 = jnp.clip(x0 + 1, 0, _W - 1)
    x0 = jnp.clip(x0, 0, _W - 1)
    y1 = jnp.clip(y0 + 1, 0, _H - 1)
    y0 = jnp.clip(y0, 0, _H - 1)
    x0f = x0.astype(xi.dtype)
    x1f = x1.astype(xi.dtype)
    y0f = y0.astype(yi.dtype)
    y1f = y1.astype(yi.dtype)
    mf = ((zi > -2.8) & (zi < 1.0)).astype(jnp.float32)
    wa = (x1f - xi) * (y1f - yi) * mf
    wb = (x1f - xi) * (yi - y0f) * mf
    wc = (xi - x0f) * (y1f - yi) * mf
    wd = (xi - x0f) * (yi - y0f) * mf
    cidx = jnp.stack([y0 * _W + x0, y1 * _W + x0,
                      y0 * _W + x1, y1 * _W + x1], axis=1)
    ws = jnp.stack([wa, wb, wc, wd], axis=1)

    row_lo = jnp.clip(jnp.floor(jnp.min(yi, axis=1)).astype(jnp.int32),
                      0, _H - 1)
    row_hi = jnp.clip(jnp.floor(jnp.max(yi, axis=1)).astype(jnp.int32) + 1,
                      0, _H - 1)
    rr = jnp.zeros((_B, _LANES), jnp.int32)
    rr = rr.at[:, 0].set(row_lo).at[:, 1].set(row_hi)

    ft = _sc_gather(cidx, ws, rr, spatial_features)
    return _mm_call(ft, W_fuse, gamma, beta)


# speedup vs baseline: 1.0569x; 1.0569x over previous
import numpy as np
import jax
import jax.numpy as jnp
from jax import lax
from jax.experimental import pallas as pl
from jax.experimental.pallas import tpu as pltpu
from jax.experimental.pallas import tpu_sc as plsc

_PCX = np.float32(-51.2)
_PCY = np.float32(-51.2)
_VOXX = np.float32(0.1)
_VOXY = np.float32(0.1)
_B = 2
_C = 128
_H = 512
_W = 512
_K = 4096
_CPT = 8
_RB = 16
_LANES = 16
_STEPS = _K // _LANES
_BN_DIV = np.float32(np.sqrt(np.float32(1.0) + np.float32(1e-5)))


def _floor_i32(v):
    t = v.astype(jnp.int32)
    tf = t.astype(jnp.float32)
    return jnp.where(tf > v, t - 1, t)


def _sc_body(xi, yi, zi, rows_hbm, out_hbm, xv, yv, zv, chunk, acc, sem, sem2):
    b = lax.axis_index("c")
    s = lax.axis_index("s")
    c0 = s * _CPT

    dy = pltpu.async_copy(yi.at[b], yv, sem2)
    dx = pltpu.async_copy(xi.at[b], xv, sem)
    dz = pltpu.async_copy(zi.at[b], zv, sem)
    dy.wait()

    zero = jnp.zeros((_LANES,), jnp.float32)
    big = jnp.full((_LANES,), 1e30, jnp.float32)

    def scan_step(i, mm):
        vmin, vmax = mm
        y = yv[pl.ds(i * _LANES, _LANES)]
        return (jnp.minimum(vmin, y), jnp.maximum(vmax, y))

    vmin, vmax = lax.fori_loop(0, _STEPS, scan_step, (big, -big))
    dx.wait()
    dz.wait()
    mn = vmin[0]
    mx = vmax[0]
    for i in range(1, _LANES):
        mn = jnp.minimum(mn, vmin[i])
        mx = jnp.maximum(mx, vmax[i])
    ymin_f = jnp.clip(mn, -2.0, 513.0)
    ymax_f = jnp.clip(mx, -2.0, 513.0)
    row_lo = jnp.clip(_floor_i32(ymin_f), 0, _H - 1)
    row_hi = jnp.clip(_floor_i32(ymax_f) + 1, 0, _H - 1)
    nch = (row_hi - row_lo + _RB) // _RB

    chunk_ch = [chunk.at[pl.ds(ch * _RB * _W, _RB * _W)] for ch in range(_CPT)]

    def issue_chunk(row_base):
        descs = []
        for ch in range(_CPT):
            for slot in range(_RB):
                descs.append(pltpu.async_copy(
                    rows_hbm.at[b, c0 + ch, row_base + slot, :],
                    chunk.at[pl.ds((ch * _RB + slot) * _W, _W)], sem))
        return descs

    def corner_weights(i):
        sl = pl.ds(i * _LANES, _LANES)
        x = xv[sl]
        y = yv[sl]
        z = zv[sl]
        x0u = jnp.clip(x, 0.0, 513.0).astype(jnp.int32)
        y0u = jnp.clip(y, 0.0, 513.0).astype(jnp.int32)
        x0 = jnp.minimum(x0u, _W - 1)
        x1 = jnp.minimum(x0u + 1, _W - 1)
        y0 = jnp.minimum(y0u, _H - 1)
        y1 = jnp.minimum(y0u + 1, _H - 1)
        x0f = x0.astype(jnp.float32)
        x1f = x1.astype(jnp.float32)
        y0f = y0.astype(jnp.float32)
        y1f = y1.astype(jnp.float32)
        mf = jnp.where((z > -2.8) & (z < 1.0), 1.0, 0.0).astype(jnp.float32)
        dy1 = (y1f - y) * mf
        dy0 = (y - y0f) * mf
        wa = (x1f - x) * dy1
        wb = (x1f - x) * dy0
        wc = (x - x0f) * dy1
        wd = (x - x0f) * dy0
        return x0, x1, y0, y1, wa, wb, wc, wd

    def fast_path():
        row_base = jnp.minimum(row_lo, _H - _RB)
        descs = issue_chunk(row_base)
        for d in descs:
            d.wait()

        def substep(i):
            sl = pl.ds(i * _LANES, _LANES)
            x0, x1, y0, y1, wa, wb, wc, wd = corner_weights(i)
            r0 = (y0 - row_base) * _W
            r1 = (y1 - row_base) * _W
            accs = [None] * _CPT
            for base, w in ((r0 + x0, wa), (r1 + x0, wb),
                            (r0 + x1, wc), (r1 + x1, wd)):
                for ch in range(_CPT):
                    v = plsc.load_gather(chunk_ch[ch], [base])
                    accs[ch] = v * w if accs[ch] is None else accs[ch] + v * w
            for ch in range(_CPT):
                acc[ch, sl] = accs[ch]

        @plsc.parallel_loop(0, _STEPS, unroll=2)
        def _steps(i):
            substep(i)

    def slow_path():
        def zstep(i, carry):
            for ch in range(_CPT):
                acc[ch, pl.ds(i * _LANES, _LANES)] = zero
            return carry

        lax.fori_loop(0, _STEPS, zstep, 0)

        def chunk_body(j, carry):
            chunk_lo = row_lo + j * _RB
            row_base = jnp.minimum(chunk_lo, _H - _RB)
            descs = issue_chunk(row_base)
            for d in descs:
                d.wait()

            def step(i, carry2):
                sl = pl.ds(i * _LANES, _LANES)
                x0, x1, y0, y1, wa, wb, wc, wd = corner_weights(i)
                accs = [acc[ch, sl] for ch in range(_CPT)]
                for yr, pairs in ((y0, ((x0, wa), (x1, wc))),
                                  (y1, ((x0, wb), (x1, wd)))):
                    valid = (yr >= chunk_lo) & (yr < chunk_lo + _RB)
                    lrow = jnp.clip(yr - row_base, 0, _RB - 1)
                    for xc, w in pairs:
                        wm = jnp.where(valid, w, 0.0)
                        base = lrow * _W + xc
                        for ch in range(_CPT):
                            v = plsc.load_gather(chunk_ch[ch], [base])
                            accs[ch] = accs[ch] + wm * v
                for ch in range(_CPT):
                    acc[ch, sl] = accs[ch]
                return carry2

            lax.fori_loop(0, _STEPS, step, 0)
            return carry

        lax.fori_loop(0, nch, chunk_body, 0)

    lax.cond(nch == 1, fast_path, slow_path)
    pltpu.sync_copy(acc, out_hbm.at[b, pl.ds(c0, _CPT), :])


def _sc_gather(xi, yi, zi, rows):
    fn = pl.kernel(
        _sc_body,
        out_type=jax.ShapeDtypeStruct((_B, _C, _K), jnp.float32),
        mesh=plsc.VectorSubcoreMesh(
            core_axis_name="c", subcore_axis_name="s",
            num_cores=2, num_subcores=16),
        scratch_types=[
            pltpu.VMEM((_K,), jnp.float32),
            pltpu.VMEM((_K,), jnp.float32),
            pltpu.VMEM((_K,), jnp.float32),
            pltpu.VMEM((_CPT * _RB * _W,), jnp.float32),
            pltpu.VMEM((_CPT, _K), jnp.float32),
            pltpu.SemaphoreType.DMA,
            pltpu.SemaphoreType.DMA,
        ],
        compiler_params=pltpu.CompilerParams(
            use_tc_tiling_on_sc=True, needs_layout_passes=False),
    )
    return fn(xi, yi, zi, rows)


def _mm_body(ft_ref, w_ref, g_ref, bt_ref, o_ref):
    wmat = w_ref[...]
    g = g_ref[...]
    bt = bt_ref[...]
    for b in range(_B):
        ft = ft_ref[b]
        y = lax.dot_general(ft, wmat, (((0,), (1,)), ((), ())),
                            preferred_element_type=jnp.float32)
        y = y / _BN_DIV * g[None, :] + bt[None, :]
        o_ref[pl.ds(b * _K, _K), :] = jnp.maximum(y, 0.0)


def _mm_call(ft, W_fuse, gamma, beta):
    return pl.pallas_call(
        _mm_body,
        out_shape=jax.ShapeDtypeStruct((_B * _K, _C), jnp.float32),
    )(ft, W_fuse, gamma, beta)


def kernel(keypoints, spatial_features, W_fuse, gamma, beta, bev_stride):
    xi = (keypoints[:, :, 0] - _PCX) / _VOXX / bev_stride
    yi = (keypoints[:, :, 1] - _PCY) / _VOXY / bev_stride
    zi = keypoints[:, :, 2]
    ft = _sc_gather(xi, yi, zi, spatial_features)
    return _mm_call(ft, W_fuse, gamma, beta)
